# baseline (speedup 1.0000x reference)
import functools

import jax
import jax.numpy as jnp
from jax import lax
from jax.experimental import pallas as pl
from jax.experimental.pallas import tpu as pltpu
from jax.experimental.pallas import tpu_sc as plsc

VOCAB = 65
T = 8
BATCH = 16384
NTOK = BATCH * T

NCORES = 2
NSUB = 16
NW = NCORES * NSUB
TOK_PER_W = NTOK // NW
CHUNK = 128
NCH = TOK_PER_W // CHUNK


def _table_body(tok_ref, pos_ref, w_ref, b_ref, tw_ref, pw_ref):
    L = jnp.dot(tok_ref[...], w_ref[...], preferred_element_type=jnp.float32)
    tw_ref[...] = L + b_ref[...]
    pw_ref[...] = jnp.dot(pos_ref[...], w_ref[...], preferred_element_type=jnp.float32)


def _build_tables(tok_table, pos_table, W, b):
    return pl.pallas_call(
        _table_body,
        out_shape=(
            jax.ShapeDtypeStruct((VOCAB, VOCAB), jnp.float32),
            jax.ShapeDtypeStruct((T, VOCAB), jnp.float32),
        ),
    )(tok_table, pos_table, W, b.reshape(1, VOCAB))


_SLOTS = (0, 16, 32, 48, 49)


def _sc_body(x_hbm, tw_hbm, pw_hbm, out_hbm, tw_v, pw_v, tb1, x_v, out_v, sem):
    wid = lax.axis_index("s") * NCORES + lax.axis_index("c")
    base = wid * TOK_PER_W
    lanes = lax.broadcasted_iota(jnp.int32, (16,), 0)
    patt = (lanes & 7) * VOCAB

    pltpu.sync_copy(tw_hbm, tw_v)
    pltpu.sync_copy(pw_hbm, pw_v)

    def build(v, carry):
        for t in range(T):
            dst = (v * T + t) * VOCAB
            for c in _SLOTS:
                tb1[pl.ds(dst + c, 16)] = tw_v[v, pl.ds(c, 16)] + pw_v[t, pl.ds(c, 16)]
        return carry

    lax.fori_loop(0, VOCAB, build, 0)

    def chunk(i, carry):
        cbase = base + i * CHUNK
        pltpu.sync_copy(x_hbm.at[pl.ds(cbase, CHUNK)], x_v)

        def group(g, c2):
            xv = x_v[pl.ds(g * 16, 16)]
            rows = xv * (T * VOCAB) + patt
            toks = g * 16 + lanes
            for c in range(VOCAB):
                col = lanes * 0 + c
                vals = plsc.load_gather(tb1, [rows + c])
                plsc.store_scatter(out_v, [toks, col], vals)
            return c2

        lax.fori_loop(0, CHUNK // 16, group, 0)
        pltpu.sync_copy(out_v, out_hbm.at[pl.ds(cbase, CHUNK), :])
        return carry

    lax.fori_loop(0, NCH, chunk, 0)


@functools.partial(
    pl.kernel,
    out_type=jax.ShapeDtypeStruct((NTOK, VOCAB), jnp.float32),
    mesh=plsc.VectorSubcoreMesh(core_axis_name="c", subcore_axis_name="s"),
    scratch_types=[
        pltpu.VMEM((VOCAB, VOCAB), jnp.float32),
        pltpu.VMEM((T, VOCAB), jnp.float32),
        pltpu.VMEM((VOCAB * T * VOCAB,), jnp.float32),
        pltpu.VMEM((CHUNK,), jnp.int32),
        pltpu.VMEM((CHUNK, VOCAB), jnp.float32),
        pltpu.SemaphoreType.DMA,
    ],
    compiler_params=pltpu.CompilerParams(needs_layout_passes=False),
)
def _sc_gather(x_hbm, tw_hbm, pw_hbm, out_hbm, tw_v, pw_v, tb1, x_v, out_v, sem):
    _sc_body(x_hbm, tw_hbm, pw_hbm, out_hbm, tw_v, pw_v, tb1, x_v, out_v, sem)


@jax.jit
def kernel(x, tok_table, pos_table, W, b):
    tw, pw = _build_tables(tok_table, pos_table, W, b)
    xf = x.reshape(NTOK).astype(jnp.int32)
    out = _sc_gather(xf, tw, pw)
    return out.reshape(BATCH, T, VOCAB)
